# Initial kernel scaffold; baseline (speedup 1.0000x reference)
import jax
import jax.numpy as jnp
from jax.experimental import pallas as pl


def kernel(x, edge_index, batch, W1, b1, W2, b2, Wm, bm):
    raise NotImplementedError("write your pallas kernel here")


# speedup vs baseline: 18.1011x; 18.1011x over previous
import functools

import jax
import jax.numpy as jnp
from jax import lax
from jax.experimental import pallas as pl
from jax.experimental.pallas import tpu as pltpu
from jax.experimental.pallas import tpu_sc as plsc

N = 10000
E = 320000
D = 128
H = 128
O = 64
G = 64

NPAD = 10112
ROWS_PER_TILE = 632
CH = 128
E_PAD = 327680
ECR = E_PAD // CH
AGG_CR_PER_TILE = ECR // 16
IDX_GRP = 16
DEG_CR_PER_WORKER = ECR // 32

_MESH = dict(core_axis_name="c", subcore_axis_name="s")
_F32 = jnp.float32


def _sc_degree(dst2d):
    @functools.partial(
        pl.kernel,
        out_type=jax.ShapeDtypeStruct((2, NPAD, 16), _F32),
        mesh=plsc.VectorSubcoreMesh(**_MESH),
        scratch_types=[
            pltpu.VMEM_SHARED((NPAD, 16), _F32),
            pltpu.VMEM((DEG_CR_PER_WORKER, CH), jnp.int32),
            pltpu.VMEM((ROWS_PER_TILE, 16), _F32),
            pltpu.VMEM((CH, 16), _F32),
        ],
    )
    def deg_kernel(dst_hbm, out_hbm, acc_sh, idx_v, zb_v, ones_v):
        c = lax.axis_index("c")
        s = lax.axis_index("s")
        w = c * 16 + s
        rs = pl.ds(s * ROWS_PER_TILE, ROWS_PER_TILE)

        @pl.loop(0, ROWS_PER_TILE)
        def _(i):
            zb_v[i, :] = jnp.zeros((16,), _F32)

        @pl.loop(0, CH)
        def _(i):
            ones_v[i, :] = jnp.ones((16,), _F32)

        pltpu.sync_copy(zb_v, acc_sh.at[rs, :])
        pltpu.sync_copy(dst_hbm.at[w], idx_v)
        plsc.subcore_barrier()

        @pl.loop(0, DEG_CR_PER_WORKER)
        def _(j):
            pltpu.sync_copy(ones_v, acc_sh.at[idx_v.at[j]], add=True)

        plsc.subcore_barrier()
        pltpu.sync_copy(acc_sh.at[rs, :], out_hbm.at[c, rs, :])

    return deg_kernel(dst2d)


def _sc_aggregate(hs_split, src2d, dst2d):
    @functools.partial(
        pl.kernel,
        out_type=jax.ShapeDtypeStruct((2, NPAD, 64), _F32),
        mesh=plsc.VectorSubcoreMesh(**_MESH),
        scratch_types=[
            pltpu.VMEM_SHARED((NPAD, 64), _F32),
            pltpu.VMEM_SHARED((NPAD, 64), _F32),
            pltpu.VMEM((IDX_GRP, CH), jnp.int32),
            pltpu.VMEM((IDX_GRP, CH), jnp.int32),
            pltpu.VMEM((CH, 64), _F32),
        ],
    )
    def agg_kernel(hs_hbm, src_hbm, dst_hbm, out_hbm, tbl_sh, acc_sh, si_v, di_v, rows_v):
        c = lax.axis_index("c")
        s = lax.axis_index("s")
        rs = pl.ds(s * ROWS_PER_TILE, ROWS_PER_TILE)

        pltpu.sync_copy(hs_hbm.at[c, rs, :], tbl_sh.at[rs, :])
        pltpu.sync_copy(hs_hbm.at[c, rs, :], acc_sh.at[rs, :])
        plsc.subcore_barrier()

        @pl.loop(0, AGG_CR_PER_TILE // IDX_GRP)
        def _(g):
            gs = pl.ds(g * IDX_GRP, IDX_GRP)
            pltpu.sync_copy(src_hbm.at[s, gs, :], si_v)
            pltpu.sync_copy(dst_hbm.at[s, gs, :], di_v)

            @pl.loop(0, IDX_GRP)
            def _(j):
                pltpu.sync_copy(tbl_sh.at[si_v.at[j]], rows_v)
                pltpu.sync_copy(rows_v, acc_sh.at[di_v.at[j]], add=True)

        plsc.subcore_barrier()
        pltpu.sync_copy(acc_sh.at[rs, :], out_hbm.at[c, rs, :])

    return agg_kernel(hs_split, src2d, dst2d)


def _dinv_from_parts(dp):
    deg = dp[0] + dp[1] + 1.0
    return lax.rsqrt(deg)[:, 0:1]


def _tc_matmul(x, W):
    def body(x_ref, w_ref, o_ref):
        o_ref[...] = lax.dot_general(
            x_ref[...], w_ref[...], (((1,), (0,)), ((), ())),
            precision=lax.Precision.HIGHEST, preferred_element_type=_F32)

    return pl.pallas_call(
        body, out_shape=jax.ShapeDtypeStruct((N, H), _F32))(x, W)


def _split_pad_store(o_ref, hs):
    o_ref[0, :N, :] = hs[:, :64]
    o_ref[1, :N, :] = hs[:, 64:]
    zpad = jnp.zeros((NPAD - N, 64), _F32)
    o_ref[0, N:, :] = zpad
    o_ref[1, N:, :] = zpad


def _tc_scale_split(xW, degp):
    def body(xw_ref, dp_ref, o_ref):
        dinv = _dinv_from_parts(dp_ref[...])
        _split_pad_store(o_ref, xw_ref[...] * dinv[:N])

    return pl.pallas_call(
        body, out_shape=jax.ShapeDtypeStruct((2, NPAD, 64), _F32))(xW, degp)


def _tc_layer(agg, degp, b, W):
    def body(agg_ref, dp_ref, b_ref, w_ref, o_ref):
        dinv = _dinv_from_parts(dp_ref[...])
        h = jnp.concatenate([agg_ref[0, :N, :], agg_ref[1, :N, :]], axis=1)
        h = jnp.maximum(h * dinv[:N] + b_ref[...], 0.0)
        hs = lax.dot_general(
            h, w_ref[...], (((1,), (0,)), ((), ())),
            precision=lax.Precision.HIGHEST, preferred_element_type=_F32)
        _split_pad_store(o_ref, hs * dinv[:N])

    return pl.pallas_call(
        body, out_shape=jax.ShapeDtypeStruct((2, NPAD, 64), _F32))(agg, degp, b, W)


def _tc_head(agg, degp, b, batch, Wm, bm):
    def body(agg_ref, dp_ref, b_ref, bt_ref, wm_ref, bm_ref, out_ref, gx_ref):
        dinv = _dinv_from_parts(dp_ref[...])
        h = jnp.concatenate([agg_ref[0, :N, :], agg_ref[1, :N, :]], axis=1)
        h = h * dinv[:N] + b_ref[...]
        onehot = (bt_ref[...][:, None]
                  == lax.broadcasted_iota(jnp.int32, (N, G), 1)).astype(_F32)
        counts = jnp.sum(onehot, axis=0)
        pooled = lax.dot_general(
            onehot, h, (((0,), (0,)), ((), ())),
            precision=lax.Precision.HIGHEST, preferred_element_type=_F32)
        gx = pooled / jnp.maximum(counts, 1.0)[:, None]
        out = lax.dot_general(
            gx, wm_ref[...], (((1,), (0,)), ((), ())),
            precision=lax.Precision.HIGHEST, preferred_element_type=_F32)
        out_ref[...] = out + bm_ref[...]
        gx_ref[...] = gx

    return pl.pallas_call(
        body,
        out_shape=(jax.ShapeDtypeStruct((G, O), _F32),
                   jax.ShapeDtypeStruct((G, H), _F32)),
    )(agg, degp, b, batch, Wm, bm)


def kernel(x, edge_index, batch, W1, b1, W2, b2, Wm, bm):
    pad = N + (jnp.arange(E_PAD - E, dtype=jnp.int32) % 16)
    src_p = jnp.concatenate([edge_index[0], pad])
    dst_p = jnp.concatenate([edge_index[1], pad])
    src3d = src_p.reshape(16, AGG_CR_PER_TILE, CH)
    dst3d = dst_p.reshape(16, AGG_CR_PER_TILE, CH)
    dst_deg = dst_p.reshape(32, DEG_CR_PER_WORKER, CH)

    degp = _sc_degree(dst_deg)
    xW1 = _tc_matmul(x, W1)
    hs1 = _tc_scale_split(xW1, degp)
    agg1 = _sc_aggregate(hs1, src3d, dst3d)
    hs2 = _tc_layer(agg1, degp, b1, W2)
    agg2 = _sc_aggregate(hs2, src3d, dst3d)
    return _tc_head(agg2, degp, b2, batch, Wm, bm)


# speedup vs baseline: 21.7569x; 1.2020x over previous
import functools

import jax
import jax.numpy as jnp
from jax import lax
from jax.experimental import pallas as pl
from jax.experimental.pallas import tpu as pltpu
from jax.experimental.pallas import tpu_sc as plsc

N = 10000
E = 320000
D = 128
H = 128
O = 64
G = 64

NPAD = 10112
ROWS_PER_TILE = 632
CH = 128
E_PAD = 327680
ECR = E_PAD // CH
AGG_CR_PER_TILE = ECR // 16
IDX_GRP = 16
DEG_CR_PER_WORKER = ECR // 32

_MESH = dict(core_axis_name="c", subcore_axis_name="s")
_F32 = jnp.float32


def _sc_degree(dst2d):
    @functools.partial(
        pl.kernel,
        out_type=jax.ShapeDtypeStruct((2, NPAD, 16), _F32),
        mesh=plsc.VectorSubcoreMesh(**_MESH),
        scratch_types=[
            pltpu.VMEM_SHARED((NPAD, 16), _F32),
            pltpu.VMEM((DEG_CR_PER_WORKER, CH), jnp.int32),
            pltpu.VMEM((ROWS_PER_TILE, 16), _F32),
            pltpu.VMEM((CH, 16), _F32),
        ],
    )
    def deg_kernel(dst_hbm, out_hbm, acc_sh, idx_v, zb_v, ones_v):
        c = lax.axis_index("c")
        s = lax.axis_index("s")
        w = c * 16 + s
        rs = pl.ds(s * ROWS_PER_TILE, ROWS_PER_TILE)

        @pl.loop(0, ROWS_PER_TILE)
        def _(i):
            zb_v[i, :] = jnp.zeros((16,), _F32)

        @pl.loop(0, CH)
        def _(i):
            ones_v[i, :] = jnp.ones((16,), _F32)

        pltpu.sync_copy(zb_v, acc_sh.at[rs, :])
        pltpu.sync_copy(dst_hbm.at[w], idx_v)
        plsc.subcore_barrier()

        @pl.loop(0, DEG_CR_PER_WORKER)
        def _(j):
            pltpu.sync_copy(ones_v, acc_sh.at[idx_v.at[j]], add=True)

        plsc.subcore_barrier()
        pltpu.sync_copy(acc_sh.at[rs, :], out_hbm.at[c, rs, :])

    return deg_kernel(dst2d)


def _sc_aggregate(hs_split, src2d, dst2d):
    @functools.partial(
        pl.kernel,
        out_type=jax.ShapeDtypeStruct((2, NPAD, 64), _F32),
        mesh=plsc.VectorSubcoreMesh(**_MESH),
        scratch_types=[
            pltpu.VMEM_SHARED((NPAD, 64), _F32),
            pltpu.VMEM_SHARED((NPAD, 64), _F32),
            pltpu.VMEM((IDX_GRP, CH), jnp.int32),
            pltpu.VMEM((IDX_GRP, CH), jnp.int32),
            pltpu.VMEM((CH, 64), _F32),
            pltpu.VMEM((CH, 64), _F32),
            pltpu.SemaphoreType.DMA,
            pltpu.SemaphoreType.DMA,
        ],
    )
    def agg_kernel(hs_hbm, src_hbm, dst_hbm, out_hbm, tbl_sh, acc_sh,
                   si_v, di_v, rows_a, rows_b, sem_a, sem_b):
        c = lax.axis_index("c")
        s = lax.axis_index("s")
        rs = pl.ds(s * ROWS_PER_TILE, ROWS_PER_TILE)

        pltpu.sync_copy(hs_hbm.at[c, rs, :], tbl_sh.at[rs, :])
        pltpu.sync_copy(hs_hbm.at[c, rs, :], acc_sh.at[rs, :])
        plsc.subcore_barrier()

        @pl.loop(0, AGG_CR_PER_TILE // IDX_GRP)
        def _(g):
            gs = pl.ds(g * IDX_GRP, IDX_GRP)
            pltpu.sync_copy(src_hbm.at[s, gs, :], si_v)
            pltpu.sync_copy(dst_hbm.at[s, gs, :], di_v)

            ga = pltpu.async_copy(tbl_sh.at[si_v.at[0]], rows_a, sem_a)
            for p in range(IDX_GRP // 2):
                gb = pltpu.async_copy(tbl_sh.at[si_v.at[2 * p + 1]], rows_b, sem_b)
                ga.wait()
                pltpu.sync_copy(rows_a, acc_sh.at[di_v.at[2 * p]], add=True)
                if p < IDX_GRP // 2 - 1:
                    ga = pltpu.async_copy(tbl_sh.at[si_v.at[2 * p + 2]], rows_a, sem_a)
                gb.wait()
                pltpu.sync_copy(rows_b, acc_sh.at[di_v.at[2 * p + 1]], add=True)

        plsc.subcore_barrier()
        pltpu.sync_copy(acc_sh.at[rs, :], out_hbm.at[c, rs, :])

    return agg_kernel(hs_split, src2d, dst2d)


def _dinv_from_parts(dp):
    deg = dp[0] + dp[1] + 1.0
    return lax.rsqrt(deg)[:, 0:1]


def _tc_matmul(x, W):
    def body(x_ref, w_ref, o_ref):
        o_ref[...] = lax.dot_general(
            x_ref[...], w_ref[...], (((1,), (0,)), ((), ())),
            precision=lax.Precision.HIGHEST, preferred_element_type=_F32)

    return pl.pallas_call(
        body, out_shape=jax.ShapeDtypeStruct((N, H), _F32))(x, W)


def _split_pad_store(o_ref, hs):
    o_ref[0, :N, :] = hs[:, :64]
    o_ref[1, :N, :] = hs[:, 64:]
    zpad = jnp.zeros((NPAD - N, 64), _F32)
    o_ref[0, N:, :] = zpad
    o_ref[1, N:, :] = zpad


def _tc_scale_split(xW, degp):
    def body(xw_ref, dp_ref, o_ref):
        dinv = _dinv_from_parts(dp_ref[...])
        _split_pad_store(o_ref, xw_ref[...] * dinv[:N])

    return pl.pallas_call(
        body, out_shape=jax.ShapeDtypeStruct((2, NPAD, 64), _F32))(xW, degp)


def _tc_layer(agg, degp, b, W):
    def body(agg_ref, dp_ref, b_ref, w_ref, o_ref):
        dinv = _dinv_from_parts(dp_ref[...])
        h = jnp.concatenate([agg_ref[0, :N, :], agg_ref[1, :N, :]], axis=1)
        h = jnp.maximum(h * dinv[:N] + b_ref[...], 0.0)
        hs = lax.dot_general(
            h, w_ref[...], (((1,), (0,)), ((), ())),
            precision=lax.Precision.HIGHEST, preferred_element_type=_F32)
        _split_pad_store(o_ref, hs * dinv[:N])

    return pl.pallas_call(
        body, out_shape=jax.ShapeDtypeStruct((2, NPAD, 64), _F32))(agg, degp, b, W)


def _tc_head(agg, degp, b, batch, Wm, bm):
    def body(agg_ref, dp_ref, b_ref, bt_ref, wm_ref, bm_ref, out_ref, gx_ref):
        dinv = _dinv_from_parts(dp_ref[...])
        h = jnp.concatenate([agg_ref[0, :N, :], agg_ref[1, :N, :]], axis=1)
        h = h * dinv[:N] + b_ref[...]
        onehot = (bt_ref[...][:, None]
                  == lax.broadcasted_iota(jnp.int32, (N, G), 1)).astype(_F32)
        counts = jnp.sum(onehot, axis=0)
        pooled = lax.dot_general(
            onehot, h, (((0,), (0,)), ((), ())),
            precision=lax.Precision.HIGHEST, preferred_element_type=_F32)
        gx = pooled / jnp.maximum(counts, 1.0)[:, None]
        out = lax.dot_general(
            gx, wm_ref[...], (((1,), (0,)), ((), ())),
            precision=lax.Precision.HIGHEST, preferred_element_type=_F32)
        out_ref[...] = out + bm_ref[...]
        gx_ref[...] = gx

    return pl.pallas_call(
        body,
        out_shape=(jax.ShapeDtypeStruct((G, O), _F32),
                   jax.ShapeDtypeStruct((G, H), _F32)),
    )(agg, degp, b, batch, Wm, bm)


def kernel(x, edge_index, batch, W1, b1, W2, b2, Wm, bm):
    pad = N + (jnp.arange(E_PAD - E, dtype=jnp.int32) % 16)
    src_p = jnp.concatenate([edge_index[0], pad])
    dst_p = jnp.concatenate([edge_index[1], pad])
    src3d = src_p.reshape(16, AGG_CR_PER_TILE, CH)
    dst3d = dst_p.reshape(16, AGG_CR_PER_TILE, CH)
    dst_deg = dst_p.reshape(32, DEG_CR_PER_WORKER, CH)

    degp = _sc_degree(dst_deg)
    xW1 = _tc_matmul(x, W1)
    hs1 = _tc_scale_split(xW1, degp)
    agg1 = _sc_aggregate(hs1, src3d, dst3d)
    hs2 = _tc_layer(agg1, degp, b1, W2)
    agg2 = _sc_aggregate(hs2, src3d, dst3d)
    return _tc_head(agg2, degp, b2, batch, Wm, bm)
